# baseline (speedup 1.0000x reference)
import functools

import jax
import jax.numpy as jnp
from jax import lax
from jax.experimental import pallas as pl
from jax.experimental.pallas import tpu as pltpu
from jax.experimental.pallas import tpu_sc as plsc

N = 10000
E = 320000
D = 128
B = 16
N_C = 8
D_COND = 16

NPAD = 10240
NCORES = 2
NSUB = 16
NW = NCORES * NSUB
CH = 128
NCHUNK = 80
EPAD = NW * NCHUNK * CH
RPT = NPAD // NSUB
BLK = 5120
GRID = NPAD // BLK

_HIGH = jax.lax.Precision.HIGHEST


def _sc_degree_body(eidx_hbm, out_hbm, degacc, dstv, onesv, zbuf):
    c = lax.axis_index("c")
    s = lax.axis_index("s")
    wid = c * NSUB + s
    for i in range(CH // 16):
        onesv[pl.ds(i * 16, 16)] = jnp.ones((16,), jnp.float32)
    for i in range(RPT // 16):
        zbuf[pl.ds(i * 16, 16)] = jnp.zeros((16,), jnp.float32)
    pltpu.sync_copy(zbuf, degacc.at[pl.ds(s * RPT, RPT)])
    pltpu.sync_copy(eidx_hbm.at[pl.ds(wid * NCHUNK, NCHUNK)], dstv)
    plsc.subcore_barrier()

    def body(j, carry):
        pltpu.sync_copy(onesv, degacc.at[dstv.at[j, 1]], add=True)
        return carry

    lax.fori_loop(0, NCHUNK, body, 0)
    plsc.subcore_barrier()
    pltpu.sync_copy(degacc.at[pl.ds(s * RPT, RPT)],
                    out_hbm.at[pl.ds(c * NPAD + s * RPT, RPT)])


NBUF = 2
WIN = 20
NWIN = NCHUNK // WIN


def _sc_edge_body(y_hbm, eidx_hbm, out_hbm,
                  accS, ew, rows, s0, s1, si):
    c = lax.axis_index("c")
    s = lax.axis_index("s")
    wid = c * NSUB + s
    r0 = s * RPT
    sems = (s0, s1)
    slab = wid * NCHUNK

    pltpu.async_copy(y_hbm.at[pl.ds(r0, RPT)], accS.at[pl.ds(r0, RPT)], s0)
    pltpu.sync_copy(eidx_hbm.at[pl.ds(slab, WIN)], ew.at[0])
    pltpu.make_async_copy(y_hbm.at[pl.ds(r0, RPT)],
                          accS.at[pl.ds(r0, RPT)], s0).wait()
    plsc.subcore_barrier()

    pltpu.async_copy(y_hbm.at[ew.at[0, 0, 0]], rows.at[0], s0)
    for j in range(NCHUNK):
        g, jj = divmod(j, WIN)
        wb = g % 2
        if jj == WIN // 2 and g + 1 < NWIN:
            pltpu.async_copy(
                eidx_hbm.at[pl.ds(slab + (g + 1) * WIN, WIN)],
                ew.at[(g + 1) % 2], si)
        if j + 1 < NCHUNK:
            g1, jj1 = divmod(j + 1, WIN)
            if jj1 == 0:
                pltpu.make_async_copy(
                    eidx_hbm.at[pl.ds(slab + g1 * WIN, WIN)],
                    ew.at[g1 % 2], si).wait()
            pltpu.async_copy(y_hbm.at[ew.at[g1 % 2, jj1, 0]],
                             rows.at[(j + 1) % NBUF], sems[(j + 1) % NBUF])
        pltpu.make_async_copy(y_hbm.at[ew.at[wb, jj, 0]],
                              rows.at[j % NBUF], sems[j % NBUF]).wait()
        pltpu.sync_copy(rows.at[j % NBUF], accS.at[ew.at[wb, jj, 1]],
                        add=True)
    plsc.subcore_barrier()
    pltpu.sync_copy(accS.at[pl.ds(r0, RPT)], out_hbm.at[c, pl.ds(r0, RPT)])


@functools.lru_cache(maxsize=None)
def _build_sc_kernels():
    mesh = plsc.VectorSubcoreMesh(core_axis_name="c", subcore_axis_name="s",
                                  num_cores=NCORES, num_subcores=NSUB)
    sc_degree = pl.kernel(
        _sc_degree_body,
        out_type=jax.ShapeDtypeStruct((NCORES * NPAD,), jnp.float32),
        mesh=mesh,
        scratch_types=[
            pltpu.VMEM_SHARED((NPAD,), jnp.float32),
            pltpu.VMEM((NCHUNK, 2, CH), jnp.int32),
            pltpu.VMEM((CH,), jnp.float32),
            pltpu.VMEM((RPT,), jnp.float32),
        ],
    )
    sc_edge = pl.kernel(
        _sc_edge_body,
        out_type=jax.ShapeDtypeStruct((NCORES, NPAD, D), jnp.float32),
        mesh=mesh,
        scratch_types=[
            pltpu.VMEM_SHARED((NPAD, D), jnp.float32),
            pltpu.VMEM((2, WIN, 2, CH), jnp.int32),
            pltpu.VMEM((NBUF, CH, D), jnp.float32),
            pltpu.SemaphoreType.DMA,
            pltpu.SemaphoreType.DMA,
            pltpu.SemaphoreType.DMA,
        ],
    )
    return sc_degree, sc_edge


def _tc_pre_body(x_ref, w_ref, dt_ref, y_ref):
    dis = lax.rsqrt(dt_ref[...])
    y_ref[...] = jnp.dot(x_ref[...], w_ref[...],
                         preferred_element_type=jnp.float32,
                         precision=_HIGH) * dis


def _tc_mid_body(acc_ref, yin_ref, dt_ref, b_ref, w_ref, y_ref):
    i = pl.program_id(0)
    dis = lax.rsqrt(dt_ref[...])
    a = acc_ref[0] + acc_ref[1] - yin_ref[...]
    h = jnp.maximum(a * dis + b_ref[...], 0.0)
    y = jnp.dot(h, w_ref[...], preferred_element_type=jnp.float32,
                precision=_HIGH) * dis
    rows = i * BLK + lax.broadcasted_iota(jnp.int32, (BLK, 1), 0)
    y_ref[...] = jnp.where(rows < N, y, 0.0)


def _tc_final_body(acc_ref, yin_ref, dt_ref, b_ref, batch_ref, cond_ref,
                   wl_ref, bl_ref, out_ref, sums, counts):
    i = pl.program_id(0)

    @pl.when(i == 0)
    def _():
        sums[...] = jnp.zeros_like(sums)
        counts[...] = jnp.zeros_like(counts)

    dis = lax.rsqrt(dt_ref[...])
    h = ((acc_ref[0] + acc_ref[1] - yin_ref[...]) * dis
         + b_ref[...])
    oh = (batch_ref[...] == lax.broadcasted_iota(jnp.int32, (1, B), 1))
    oh = oh.astype(jnp.float32)
    sums[...] += lax.dot_general(oh, h, (((0,), (0,)), ((), ())),
                                 preferred_element_type=jnp.float32,
                                 precision=_HIGH)
    counts[...] += lax.dot_general(oh, jnp.ones((BLK, 1), jnp.float32),
                                   (((0,), (0,)), ((), ())),
                                   preferred_element_type=jnp.float32,
                                   precision=_HIGH)

    @pl.when(i == pl.num_programs(0) - 1)
    def _():
        pooled = sums[...] / jnp.maximum(counts[...], 1.0)
        wl = wl_ref[...]
        out_ref[...] = (
            jnp.dot(pooled, wl[0:D, :], preferred_element_type=jnp.float32,
                    precision=_HIGH)
            + jnp.dot(cond_ref[...], wl[D:D + D_COND, :],
                      preferred_element_type=jnp.float32, precision=_HIGH)
            + bl_ref[...])


_tc_pre = pl.pallas_call(
    _tc_pre_body,
    grid=(GRID,),
    in_specs=[
        pl.BlockSpec((BLK, D), lambda i: (i, 0)),
        pl.BlockSpec((D, D), lambda i: (0, 0)),
        pl.BlockSpec((BLK, 1), lambda i: (i, 0)),
    ],
    out_specs=pl.BlockSpec((BLK, D), lambda i: (i, 0)),
    out_shape=jax.ShapeDtypeStruct((NPAD, D), jnp.float32),
)

_tc_mid = pl.pallas_call(
    _tc_mid_body,
    grid=(GRID,),
    in_specs=[
        pl.BlockSpec((NCORES, BLK, D), lambda i: (0, i, 0)),
        pl.BlockSpec((BLK, D), lambda i: (i, 0)),
        pl.BlockSpec((BLK, 1), lambda i: (i, 0)),
        pl.BlockSpec((1, D), lambda i: (0, 0)),
        pl.BlockSpec((D, D), lambda i: (0, 0)),
    ],
    out_specs=pl.BlockSpec((BLK, D), lambda i: (i, 0)),
    out_shape=jax.ShapeDtypeStruct((NPAD, D), jnp.float32),
)

_tc_final = pl.pallas_call(
    _tc_final_body,
    grid=(GRID,),
    in_specs=[
        pl.BlockSpec((NCORES, BLK, D), lambda i: (0, i, 0)),
        pl.BlockSpec((BLK, D), lambda i: (i, 0)),
        pl.BlockSpec((BLK, 1), lambda i: (i, 0)),
        pl.BlockSpec((1, D), lambda i: (0, 0)),
        pl.BlockSpec((BLK, 1), lambda i: (i, 0)),
        pl.BlockSpec((B, D_COND), lambda i: (0, 0)),
        pl.BlockSpec((D + D_COND, N_C), lambda i: (0, 0)),
        pl.BlockSpec((1, N_C), lambda i: (0, 0)),
    ],
    out_specs=pl.BlockSpec((B, N_C), lambda i: (0, 0)),
    out_shape=jax.ShapeDtypeStruct((B, N_C), jnp.float32),
    scratch_shapes=[
        pltpu.VMEM((B, D), jnp.float32),
        pltpu.VMEM((B, 1), jnp.float32),
    ],
    compiler_params=pltpu.CompilerParams(
        dimension_semantics=("arbitrary",)),
)


def kernel(x, edge_index, batch, cond, W1, b1, W2, b2, W3, b3, W4, b4,
           Wl, bl):
    x_pad = jnp.pad(x, ((0, NPAD - N), (0, 0)))
    e3 = edge_index.reshape(2, E // CH, CH).transpose(1, 0, 2)
    padrow = N + jnp.arange(CH, dtype=jnp.int32)
    pad3 = jnp.broadcast_to(padrow[None, None, :],
                            (NW * NCHUNK - E // CH, 2, CH))
    eidx = jnp.concatenate([e3, pad3], axis=0)
    batch_pad = jnp.concatenate(
        [batch, jnp.full((NPAD - N,), B, jnp.int32)]).reshape(NPAD, 1)

    _sc_degree, _sc_edge = _build_sc_kernels()
    degs = _sc_degree(eidx)
    dtot = (degs[:NPAD] + degs[NPAD:] + 1.0).reshape(NPAD, 1)

    y = _tc_pre(x_pad, W1, dtot)
    for b_k, W_next in ((b1, W2), (b2, W3), (b3, W4)):
        acc = _sc_edge(y, eidx)
        y = _tc_mid(acc, y, dtot, b_k.reshape(1, D), W_next)
    acc = _sc_edge(y, eidx)
    return _tc_final(acc, y, dtot, b4.reshape(1, D), batch_pad, cond,
                     Wl.reshape(D + D_COND, N_C), bl.reshape(1, N_C))


# speedup vs baseline: 1.0078x; 1.0078x over previous
import functools

import jax
import jax.numpy as jnp
from jax import lax
from jax.experimental import pallas as pl
from jax.experimental.pallas import tpu as pltpu
from jax.experimental.pallas import tpu_sc as plsc

N = 10000
E = 320000
D = 128
B = 16
N_C = 8
D_COND = 16

NPAD = 10240
NCORES = 2
NSUB = 16
NW = NCORES * NSUB
CH = 128
NCHUNK = 80
EPAD = NW * NCHUNK * CH
RPT = NPAD // NSUB
BLK = 2560
GRID = NPAD // BLK

_HIGH = jax.lax.Precision.HIGHEST


def _sc_degree_body(eidx_hbm, out_hbm, degacc, dstv, onesv, zbuf):
    c = lax.axis_index("c")
    s = lax.axis_index("s")
    wid = c * NSUB + s
    for i in range(CH // 16):
        onesv[pl.ds(i * 16, 16)] = jnp.ones((16,), jnp.float32)
    for i in range(RPT // 16):
        zbuf[pl.ds(i * 16, 16)] = jnp.zeros((16,), jnp.float32)
    pltpu.sync_copy(zbuf, degacc.at[pl.ds(s * RPT, RPT)])
    pltpu.sync_copy(eidx_hbm.at[pl.ds(wid * NCHUNK, NCHUNK)], dstv)
    plsc.subcore_barrier()

    def body(j, carry):
        pltpu.sync_copy(onesv, degacc.at[dstv.at[j, 1]], add=True)
        return carry

    lax.fori_loop(0, NCHUNK, body, 0)
    plsc.subcore_barrier()
    pltpu.sync_copy(degacc.at[pl.ds(s * RPT, RPT)],
                    out_hbm.at[pl.ds(c * NPAD + s * RPT, RPT)])


NBUF = 2
WIN = 20
NWIN = NCHUNK // WIN


def _sc_edge_body(y_hbm, eidx_hbm, out_hbm,
                  accS, ew, rows, s0, s1, si):
    c = lax.axis_index("c")
    s = lax.axis_index("s")
    wid = c * NSUB + s
    r0 = s * RPT
    sems = (s0, s1)
    slab = wid * NCHUNK

    pltpu.async_copy(y_hbm.at[pl.ds(r0, RPT)], accS.at[pl.ds(r0, RPT)], s0)
    pltpu.sync_copy(eidx_hbm.at[pl.ds(slab, WIN)], ew.at[0])
    pltpu.make_async_copy(y_hbm.at[pl.ds(r0, RPT)],
                          accS.at[pl.ds(r0, RPT)], s0).wait()
    plsc.subcore_barrier()

    pltpu.async_copy(y_hbm.at[ew.at[0, 0, 0]], rows.at[0], s0)
    for j in range(NCHUNK):
        g, jj = divmod(j, WIN)
        wb = g % 2
        if jj == WIN // 2 and g + 1 < NWIN:
            pltpu.async_copy(
                eidx_hbm.at[pl.ds(slab + (g + 1) * WIN, WIN)],
                ew.at[(g + 1) % 2], si)
        if j + 1 < NCHUNK:
            g1, jj1 = divmod(j + 1, WIN)
            if jj1 == 0:
                pltpu.make_async_copy(
                    eidx_hbm.at[pl.ds(slab + g1 * WIN, WIN)],
                    ew.at[g1 % 2], si).wait()
            pltpu.async_copy(y_hbm.at[ew.at[g1 % 2, jj1, 0]],
                             rows.at[(j + 1) % NBUF], sems[(j + 1) % NBUF])
        pltpu.make_async_copy(y_hbm.at[ew.at[wb, jj, 0]],
                              rows.at[j % NBUF], sems[j % NBUF]).wait()
        pltpu.sync_copy(rows.at[j % NBUF], accS.at[ew.at[wb, jj, 1]],
                        add=True)
    plsc.subcore_barrier()
    pltpu.sync_copy(accS.at[pl.ds(r0, RPT)], out_hbm.at[c, pl.ds(r0, RPT)])


@functools.lru_cache(maxsize=None)
def _build_sc_kernels():
    mesh = plsc.VectorSubcoreMesh(core_axis_name="c", subcore_axis_name="s",
                                  num_cores=NCORES, num_subcores=NSUB)
    sc_degree = pl.kernel(
        _sc_degree_body,
        out_type=jax.ShapeDtypeStruct((NCORES * NPAD,), jnp.float32),
        mesh=mesh,
        scratch_types=[
            pltpu.VMEM_SHARED((NPAD,), jnp.float32),
            pltpu.VMEM((NCHUNK, 2, CH), jnp.int32),
            pltpu.VMEM((CH,), jnp.float32),
            pltpu.VMEM((RPT,), jnp.float32),
        ],
    )
    sc_edge = pl.kernel(
        _sc_edge_body,
        out_type=jax.ShapeDtypeStruct((NCORES, NPAD, D), jnp.float32),
        mesh=mesh,
        scratch_types=[
            pltpu.VMEM_SHARED((NPAD, D), jnp.float32),
            pltpu.VMEM((2, WIN, 2, CH), jnp.int32),
            pltpu.VMEM((NBUF, CH, D), jnp.float32),
            pltpu.SemaphoreType.DMA,
            pltpu.SemaphoreType.DMA,
            pltpu.SemaphoreType.DMA,
        ],
    )
    return sc_degree, sc_edge


def _tc_xw_body(x_ref, w_ref, xw_ref):
    xw_ref[...] = jnp.dot(x_ref[...], w_ref[...],
                          preferred_element_type=jnp.float32,
                          precision=_HIGH)


def _tc_scale_body(xw_ref, dt_ref, y_ref):
    y_ref[...] = xw_ref[...] * lax.rsqrt(dt_ref[...])


def _tc_mid_body(acc_ref, yin_ref, dt_ref, b_ref, w_ref, y_ref):
    i = pl.program_id(0)
    dis = lax.rsqrt(dt_ref[...])
    a = acc_ref[0] + acc_ref[1] - yin_ref[...]
    h = jnp.maximum(a * dis + b_ref[...], 0.0)
    y = jnp.dot(h, w_ref[...], preferred_element_type=jnp.float32,
                precision=_HIGH) * dis
    rows = i * BLK + lax.broadcasted_iota(jnp.int32, (BLK, 1), 0)
    y_ref[...] = jnp.where(rows < N, y, 0.0)


def _tc_final_body(acc_ref, yin_ref, dt_ref, b_ref, batch_ref, cond_ref,
                   wl_ref, bl_ref, out_ref, sums, counts):
    i = pl.program_id(0)

    @pl.when(i == 0)
    def _():
        sums[...] = jnp.zeros_like(sums)
        counts[...] = jnp.zeros_like(counts)

    dis = lax.rsqrt(dt_ref[...])
    h = ((acc_ref[0] + acc_ref[1] - yin_ref[...]) * dis
         + b_ref[...])
    oh = (batch_ref[...] == lax.broadcasted_iota(jnp.int32, (1, B), 1))
    oh = oh.astype(jnp.float32)
    sums[...] += lax.dot_general(oh, h, (((0,), (0,)), ((), ())),
                                 preferred_element_type=jnp.float32,
                                 precision=_HIGH)
    counts[...] += lax.dot_general(oh, jnp.ones((BLK, 1), jnp.float32),
                                   (((0,), (0,)), ((), ())),
                                   preferred_element_type=jnp.float32,
                                   precision=_HIGH)

    @pl.when(i == pl.num_programs(0) - 1)
    def _():
        pooled = sums[...] / jnp.maximum(counts[...], 1.0)
        wl = wl_ref[...]
        out_ref[...] = (
            jnp.dot(pooled, wl[0:D, :], preferred_element_type=jnp.float32,
                    precision=_HIGH)
            + jnp.dot(cond_ref[...], wl[D:D + D_COND, :],
                      preferred_element_type=jnp.float32, precision=_HIGH)
            + bl_ref[...])


_tc_xw = pl.pallas_call(
    _tc_xw_body,
    grid=(GRID,),
    in_specs=[
        pl.BlockSpec((BLK, D), lambda i: (i, 0)),
        pl.BlockSpec((D, D), lambda i: (0, 0)),
    ],
    out_specs=pl.BlockSpec((BLK, D), lambda i: (i, 0)),
    out_shape=jax.ShapeDtypeStruct((NPAD, D), jnp.float32),
)

_tc_scale = pl.pallas_call(
    _tc_scale_body,
    grid=(GRID,),
    in_specs=[
        pl.BlockSpec((BLK, D), lambda i: (i, 0)),
        pl.BlockSpec((BLK, 1), lambda i: (i, 0)),
    ],
    out_specs=pl.BlockSpec((BLK, D), lambda i: (i, 0)),
    out_shape=jax.ShapeDtypeStruct((NPAD, D), jnp.float32),
)

_tc_mid = pl.pallas_call(
    _tc_mid_body,
    grid=(GRID,),
    in_specs=[
        pl.BlockSpec((NCORES, BLK, D), lambda i: (0, i, 0)),
        pl.BlockSpec((BLK, D), lambda i: (i, 0)),
        pl.BlockSpec((BLK, 1), lambda i: (i, 0)),
        pl.BlockSpec((1, D), lambda i: (0, 0)),
        pl.BlockSpec((D, D), lambda i: (0, 0)),
    ],
    out_specs=pl.BlockSpec((BLK, D), lambda i: (i, 0)),
    out_shape=jax.ShapeDtypeStruct((NPAD, D), jnp.float32),
)

_tc_final = pl.pallas_call(
    _tc_final_body,
    grid=(GRID,),
    in_specs=[
        pl.BlockSpec((NCORES, BLK, D), lambda i: (0, i, 0)),
        pl.BlockSpec((BLK, D), lambda i: (i, 0)),
        pl.BlockSpec((BLK, 1), lambda i: (i, 0)),
        pl.BlockSpec((1, D), lambda i: (0, 0)),
        pl.BlockSpec((BLK, 1), lambda i: (i, 0)),
        pl.BlockSpec((B, D_COND), lambda i: (0, 0)),
        pl.BlockSpec((D + D_COND, N_C), lambda i: (0, 0)),
        pl.BlockSpec((1, N_C), lambda i: (0, 0)),
    ],
    out_specs=pl.BlockSpec((B, N_C), lambda i: (0, 0)),
    out_shape=jax.ShapeDtypeStruct((B, N_C), jnp.float32),
    scratch_shapes=[
        pltpu.VMEM((B, D), jnp.float32),
        pltpu.VMEM((B, 1), jnp.float32),
    ],
    compiler_params=pltpu.CompilerParams(
        dimension_semantics=("arbitrary",)),
)


def kernel(x, edge_index, batch, cond, W1, b1, W2, b2, W3, b3, W4, b4,
           Wl, bl):
    x_pad = jnp.pad(x, ((0, NPAD - N), (0, 0)))
    e3 = edge_index.reshape(2, E // CH, CH).transpose(1, 0, 2)
    padrow = N + jnp.arange(CH, dtype=jnp.int32)
    pad3 = jnp.broadcast_to(padrow[None, None, :],
                            (NW * NCHUNK - E // CH, 2, CH))
    eidx = jnp.concatenate([e3, pad3], axis=0)
    batch_pad = jnp.concatenate(
        [batch, jnp.full((NPAD - N,), B, jnp.int32)]).reshape(NPAD, 1)

    _sc_degree, _sc_edge = _build_sc_kernels()
    xw = _tc_xw(x_pad, W1)
    degs = _sc_degree(eidx)
    dtot = (degs[:NPAD] + degs[NPAD:] + 1.0).reshape(NPAD, 1)

    y = _tc_scale(xw, dtot)
    for b_k, W_next in ((b1, W2), (b2, W3), (b3, W4)):
        acc = _sc_edge(y, eidx)
        y = _tc_mid(acc, y, dtot, b_k.reshape(1, D), W_next)
    acc = _sc_edge(y, eidx)
    return _tc_final(acc, y, dtot, b4.reshape(1, D), batch_pad, cond,
                     Wl.reshape(D + D_COND, N_C), bl.reshape(1, N_C))
